# Initial kernel scaffold; baseline (speedup 1.0000x reference)
import jax
import jax.numpy as jnp
from jax.experimental import pallas as pl


def kernel(x, edge_index, batch, Wl1, bl1, Wr1, Wl2, bl2, Wr2, fcW, fcb):
    raise NotImplementedError("write your pallas kernel here")


# speedup vs baseline: 3.7847x; 3.7847x over previous
import jax
import jax.numpy as jnp
from jax import lax
from jax.experimental import pallas as pl
from jax.experimental.pallas import tpu as pltpu
from jax.experimental.pallas import tpu_sc as plsc

N = 10000
E = 320000
D = 128
H = 256
C = 10
G = 16

NC = 2
NS = 16
NW = NC * NS
K = 80
NPAD = 10240
RPT = NPAD // NS
SB = 64
EPT1 = E // NW
EPT2 = E // NS


def _agg1_body(x_hbm, src_hbm, dst_hbm, zf_hbm, out_hbm,
               acc_sh, sidx, didx, rows, fbuf, sem):
    c = lax.axis_index("c")
    s = lax.axis_index("s")
    r0 = s * RPT
    pltpu.sync_copy(zf_hbm, fbuf)
    for j in range(RPT // SB):
        pltpu.sync_copy(fbuf, acc_sh.at[pl.ds(r0 + j * SB, SB)])
    plsc.subcore_barrier()
    base = (c * NS + s) * EPT1

    def chunk(i, carry):
        off = base + i * K
        pltpu.sync_copy(src_hbm.at[pl.ds(off, K)], sidx)
        pltpu.sync_copy(dst_hbm.at[pl.ds(off, K)], didx)
        pltpu.async_copy(x_hbm.at[sidx], rows, sem).wait()
        pltpu.sync_copy(rows, acc_sh.at[didx], add=True)
        return carry

    lax.fori_loop(0, EPT1 // K, chunk, 0)
    plsc.subcore_barrier()
    for j in range(RPT // SB):
        pltpu.sync_copy(acc_sh.at[pl.ds(r0 + j * SB, SB)], fbuf)
        pltpu.sync_copy(fbuf, out_hbm.at[c, pl.ds(r0 + j * SB, SB)])


def _cnt_body(dst_hbm, ones_hbm, zf_hbm, out_hbm,
              cnt_sh, didx, ones_v, fbuf):
    c = lax.axis_index("c")
    s = lax.axis_index("s")
    r0 = s * RPT
    pltpu.sync_copy(zf_hbm, fbuf)
    for j in range(RPT // SB):
        pltpu.sync_copy(fbuf, cnt_sh.at[pl.ds(r0 + j * SB, SB)])
    pltpu.sync_copy(ones_hbm, ones_v)
    plsc.subcore_barrier()
    base = (c * NS + s) * EPT1

    def chunk(i, carry):
        pltpu.sync_copy(dst_hbm.at[pl.ds(base + i * K, K)], didx)
        pltpu.sync_copy(ones_v, cnt_sh.at[didx], add=True)
        return carry

    lax.fori_loop(0, EPT1 // K, chunk, 0)
    plsc.subcore_barrier()
    for j in range(RPT // SB):
        pltpu.sync_copy(cnt_sh.at[pl.ds(r0 + j * SB, SB)], fbuf)
        pltpu.sync_copy(fbuf, out_hbm.at[c, pl.ds(r0 + j * SB, SB)])


def _agg2_body(t_hbm, src2_hbm, dst_hbm, zf_hbm, out_hbm,
               acc_sh, sidx, didx, rows, fbuf, sem):
    c = lax.axis_index("c")
    s = lax.axis_index("s")
    r0 = s * RPT
    pltpu.sync_copy(zf_hbm, fbuf)
    for j in range(RPT // SB):
        pltpu.sync_copy(fbuf, acc_sh.at[pl.ds(r0 + j * SB, SB)])
    plsc.subcore_barrier()
    base = c * E + s * EPT2
    base_d = s * EPT2

    def chunk(i, carry):
        off = base + i * K
        pltpu.sync_copy(src2_hbm.at[pl.ds(off, K)], sidx)
        pltpu.sync_copy(dst_hbm.at[pl.ds(base_d + i * K, K)], didx)
        pltpu.async_copy(t_hbm.at[sidx], rows, sem).wait()
        pltpu.sync_copy(rows, acc_sh.at[didx], add=True)
        return carry

    lax.fori_loop(0, EPT2 // K, chunk, 0)
    plsc.subcore_barrier()
    for j in range(RPT // SB):
        pltpu.sync_copy(acc_sh.at[pl.ds(r0 + j * SB, SB)], fbuf)
        pltpu.sync_copy(fbuf, out_hbm.at[c, pl.ds(r0 + j * SB, SB)])


_SC_MESH = plsc.VectorSubcoreMesh(core_axis_name="c", subcore_axis_name="s",
                                  num_cores=NC, num_subcores=NS)

_agg1 = pl.kernel(
    _agg1_body,
    out_type=jax.ShapeDtypeStruct((NC, NPAD, D), jnp.float32),
    mesh=_SC_MESH,
    scratch_types=[
        pltpu.VMEM_SHARED((NPAD, D), jnp.float32),
        pltpu.VMEM((K,), jnp.int32),
        pltpu.VMEM((K,), jnp.int32),
        pltpu.VMEM((K, D), jnp.float32),
        pltpu.VMEM((SB, D), jnp.float32),
        pltpu.SemaphoreType.DMA,
    ],
)

_cnt = pl.kernel(
    _cnt_body,
    out_type=jax.ShapeDtypeStruct((NC, NPAD, 128), jnp.float32),
    mesh=_SC_MESH,
    scratch_types=[
        pltpu.VMEM_SHARED((NPAD, 128), jnp.float32),
        pltpu.VMEM((K,), jnp.int32),
        pltpu.VMEM((K, 128), jnp.float32),
        pltpu.VMEM((SB, 128), jnp.float32),
    ],
)

_agg2 = pl.kernel(
    _agg2_body,
    out_type=jax.ShapeDtypeStruct((NC, NPAD, D), jnp.float32),
    mesh=_SC_MESH,
    scratch_types=[
        pltpu.VMEM_SHARED((NPAD, D), jnp.float32),
        pltpu.VMEM((K,), jnp.int32),
        pltpu.VMEM((K,), jnp.int32),
        pltpu.VMEM((K, D), jnp.float32),
        pltpu.VMEM((SB, D), jnp.float32),
        pltpu.SemaphoreType.DMA,
    ],
)

R = 400


def _tc1_body(acc_ref, cnt_ref, x_ref, wl_ref, wr_ref, b_ref, out_ref):
    ssum = acc_ref[0] + acc_ref[1]
    cnt = cnt_ref[0, :, 0] + cnt_ref[1, :, 0]
    rc = 1.0 / jnp.maximum(cnt, 1.0)
    mean = ssum * rc[:, None]
    h = jnp.dot(mean, wl_ref[...], preferred_element_type=jnp.float32)
    h += jnp.dot(x_ref[...], wr_ref[...], preferred_element_type=jnp.float32)
    h += b_ref[...]
    h = jnp.maximum(h, 0.0)
    out_ref[0] = h[:, :D]
    out_ref[1] = h[:, D:]


_tc1 = pl.pallas_call(
    _tc1_body,
    grid=(N // R,),
    in_specs=[
        pl.BlockSpec((NC, R, D), lambda i: (0, i, 0)),
        pl.BlockSpec((NC, R, 128), lambda i: (0, i, 0)),
        pl.BlockSpec((R, D), lambda i: (i, 0)),
        pl.BlockSpec((D, H), lambda i: (0, 0)),
        pl.BlockSpec((D, H), lambda i: (0, 0)),
        pl.BlockSpec((1, H), lambda i: (0, 0)),
    ],
    out_specs=pl.BlockSpec((NC, R, D), lambda i: (0, i, 0)),
    out_shape=jax.ShapeDtypeStruct((NC, N, D), jnp.float32),
)


def _tc2_body(acc_ref, cnt_ref, h1_ref, batch_ref, wl_ref, wr_ref, b_ref,
              fcw_ref, fcb_ref, out_ref, pooled, cntg):
    i = pl.program_id(0)
    cnt = cnt_ref[0, :, 0] + cnt_ref[1, :, 0]
    rc = 1.0 / jnp.maximum(cnt, 1.0)
    wl = wl_ref[...]
    wr = wr_ref[...]
    h2 = jnp.dot(acc_ref[0] * rc[:, None], wl[:D],
                 preferred_element_type=jnp.float32)
    h2 += jnp.dot(acc_ref[1] * rc[:, None], wl[D:],
                  preferred_element_type=jnp.float32)
    h2 += jnp.dot(h1_ref[0], wr[:D], preferred_element_type=jnp.float32)
    h2 += jnp.dot(h1_ref[1], wr[D:], preferred_element_type=jnp.float32)
    h2 += b_ref[...]
    h2 = jnp.maximum(h2, 0.0)
    b = batch_ref[0, 0]
    onehot = (b[:, None] == lax.broadcasted_iota(jnp.int32, (R, G), 1)
              ).astype(jnp.float32)
    part = lax.dot_general(onehot, h2, (((0,), (0,)), ((), ())),
                           preferred_element_type=jnp.float32)
    cg_part = jnp.sum(onehot, axis=0)[None, :]

    @pl.when(i == 0)
    def _():
        pooled[...] = jnp.zeros_like(pooled)
        cntg[...] = jnp.zeros_like(cntg)

    pooled[...] += part
    cntg[...] += cg_part

    @pl.when(i == N // R - 1)
    def _():
        rcg = 1.0 / jnp.maximum(cntg[0, :], 1.0)
        pm = pooled[...] * rcg[:, None]
        out_ref[...] = (jnp.dot(pm, fcw_ref[...],
                                preferred_element_type=jnp.float32)
                        + fcb_ref[...])


_tc2 = pl.pallas_call(
    _tc2_body,
    grid=(N // R,),
    in_specs=[
        pl.BlockSpec((NC, R, D), lambda i: (0, i, 0)),
        pl.BlockSpec((NC, R, 128), lambda i: (0, i, 0)),
        pl.BlockSpec((NC, R, D), lambda i: (0, i, 0)),
        pl.BlockSpec((1, 1, R), lambda i: (i, 0, 0)),
        pl.BlockSpec((H, H), lambda i: (0, 0)),
        pl.BlockSpec((H, H), lambda i: (0, 0)),
        pl.BlockSpec((1, H), lambda i: (0, 0)),
        pl.BlockSpec((H, C), lambda i: (0, 0)),
        pl.BlockSpec((1, C), lambda i: (0, 0)),
    ],
    out_specs=pl.BlockSpec((G, C), lambda i: (0, 0)),
    out_shape=jax.ShapeDtypeStruct((G, C), jnp.float32),
    scratch_shapes=[
        pltpu.VMEM((G, H), jnp.float32),
        pltpu.VMEM((1, G), jnp.float32),
    ],
)


def kernel(x, edge_index, batch, Wl1, bl1, Wr1, Wl2, bl2, Wr2, fcW, fcb):
    src = edge_index[0]
    dst = edge_index[1]
    zf = jnp.zeros((SB, D), jnp.float32)
    ones = jnp.ones((K, 128), jnp.float32)
    acc1 = _agg1(x, src, dst, zf)
    cnth = _cnt(dst, ones, zf)
    h1 = _tc1(acc1, cnth, x, Wl1, Wr1, bl1.reshape(1, H))
    src2 = jnp.concatenate([src, src + N])
    acc2 = _agg2(h1.reshape(NC * N, D), src2, dst, zf)
    out = _tc2(acc2, cnth, h1, batch.reshape(N // R, 1, R),
               Wl2, Wr2, bl2.reshape(1, H), fcW, fcb.reshape(1, C))
    return out


# speedup vs baseline: 7.7138x; 2.0382x over previous
import jax
import jax.numpy as jnp
from jax import lax
from jax.experimental import pallas as pl
from jax.experimental.pallas import tpu as pltpu
from jax.experimental.pallas import tpu_sc as plsc

N = 10000
E = 320000
D = 128
H = 256
C = 10
G = 16

NC = 2
NS = 16
NW = NC * NS
K = 40
RING = 5
NPAD = 10240
RPT = NPAD // NS
SB = 32
EPT1 = E // NW
EPT2 = E // NS


def _zero_acc(zf_hbm, fbuf, acc_sh, r0):
    pltpu.sync_copy(zf_hbm, fbuf)
    for j in range(RPT // SB):
        pltpu.sync_copy(fbuf, acc_sh.at[pl.ds(r0 + j * SB, SB)])


def _writeback(acc_sh, fbuf, out_hbm, c, r0):
    for j in range(RPT // SB):
        pltpu.sync_copy(acc_sh.at[pl.ds(r0 + j * SB, SB)], fbuf)
        pltpu.sync_copy(fbuf, out_hbm.at[c, pl.ds(r0 + j * SB, SB)])


def _agg_pipeline(t_hbm, src_hbm, dst_hbm, acc_sh, sidx, didx, rows,
                  isem, gsem, ssem, base_s, base_d, ch):
    ng = ch // RING

    def sslice(i):
        return src_hbm.at[pl.ds(base_s + i * K, K)]

    def dslice(i):
        return dst_hbm.at[pl.ds(base_d + i * K, K)]

    pltpu.async_copy(sslice(0), sidx[0], isem.at[0])
    pltpu.async_copy(dslice(0), didx[0], isem.at[0])
    pltpu.async_copy(sslice(1), sidx[1], isem.at[1])
    pltpu.async_copy(dslice(1), didx[1], isem.at[1])
    pltpu.make_async_copy(sslice(0), sidx[0], isem.at[0]).wait()
    pltpu.make_async_copy(dslice(0), didx[0], isem.at[0]).wait()
    pltpu.async_copy(t_hbm.at[sidx[0]], rows[0], gsem.at[0])

    def group(g, carry):
        for k in range(RING):
            i = g * RING + k
            b = k
            b1 = (k + 1) % RING
            b2 = (k + 2) % RING

            def wait_s():
                pltpu.make_async_copy(rows[b2], acc_sh.at[didx[b2]],
                                      ssem.at[b2]).wait()

            if k >= 3:
                wait_s()
            else:
                @pl.when(g > 0)
                def _():
                    wait_s()

            def load_idx():
                pltpu.async_copy(sslice(i + 2), sidx[b2], isem.at[b2])
                pltpu.async_copy(dslice(i + 2), didx[b2], isem.at[b2])

            if k < 3:
                load_idx()
            else:
                @pl.when(g < ng - 1)
                def _():
                    load_idx()

            def start_gather():
                pltpu.make_async_copy(sslice(i + 1), sidx[b1],
                                      isem.at[b1]).wait()
                pltpu.make_async_copy(dslice(i + 1), didx[b1],
                                      isem.at[b1]).wait()
                pltpu.async_copy(t_hbm.at[sidx[b1]], rows[b1], gsem.at[b1])

            if k < 4:
                start_gather()
            else:
                @pl.when(g < ng - 1)
                def _():
                    start_gather()

            pltpu.make_async_copy(t_hbm.at[sidx[b]], rows[b],
                                  gsem.at[b]).wait()
            pltpu.async_copy(rows[b], acc_sh.at[didx[b]], ssem.at[b],
                             add=True)
        return carry

    lax.fori_loop(0, ng, group, 0)
    for b in (2, 3, 4):
        pltpu.make_async_copy(rows[b], acc_sh.at[didx[b]], ssem.at[b]).wait()


def _agg1_body(x_hbm, src_hbm, dst_hbm, zf_hbm, out_hbm,
               acc_sh, sidx, didx, rows, fbuf, isem, gsem, ssem):
    c = lax.axis_index("c")
    s = lax.axis_index("s")
    r0 = s * RPT
    _zero_acc(zf_hbm, fbuf, acc_sh, r0)
    plsc.subcore_barrier()
    base = (c * NS + s) * EPT1
    _agg_pipeline(x_hbm, src_hbm, dst_hbm, acc_sh, sidx, didx, rows,
                  isem, gsem, ssem, base, base, EPT1 // K)
    plsc.subcore_barrier()
    _writeback(acc_sh, fbuf, out_hbm, c, r0)


def _agg2_body(t_hbm, src2_hbm, dst_hbm, zf_hbm, out_hbm,
               acc_sh, sidx, didx, rows, fbuf, isem, gsem, ssem):
    c = lax.axis_index("c")
    s = lax.axis_index("s")
    r0 = s * RPT
    _zero_acc(zf_hbm, fbuf, acc_sh, r0)
    plsc.subcore_barrier()
    _agg_pipeline(t_hbm, src2_hbm, dst_hbm, acc_sh, sidx, didx, rows,
                  isem, gsem, ssem, c * E + s * EPT2, s * EPT2, EPT2 // K)
    plsc.subcore_barrier()
    _writeback(acc_sh, fbuf, out_hbm, c, r0)


def _cnt_body(dst_hbm, ones_hbm, zf_hbm, out_hbm,
              cnt_sh, didx, ones_v, fbuf, isem, ssem):
    c = lax.axis_index("c")
    s = lax.axis_index("s")
    r0 = s * RPT
    _zero_acc(zf_hbm, fbuf, cnt_sh, r0)
    pltpu.sync_copy(ones_hbm, ones_v)
    plsc.subcore_barrier()
    base = (c * NS + s) * EPT1
    ch = EPT1 // K
    ng = ch // RING

    def dslice(i):
        return dst_hbm.at[pl.ds(base + i * K, K)]

    pltpu.async_copy(dslice(0), didx[0], isem.at[0])
    pltpu.async_copy(dslice(1), didx[1], isem.at[1])
    pltpu.make_async_copy(dslice(0), didx[0], isem.at[0]).wait()

    def group(g, carry):
        for k in range(RING):
            i = g * RING + k
            b = k
            b1 = (k + 1) % RING
            b2 = (k + 2) % RING

            def wait_s():
                pltpu.make_async_copy(ones_v, cnt_sh.at[didx[b2]],
                                      ssem.at[b2]).wait()

            if k >= 3:
                wait_s()
            else:
                @pl.when(g > 0)
                def _():
                    wait_s()

            def load_idx():
                pltpu.async_copy(dslice(i + 2), didx[b2], isem.at[b2])

            if k < 3:
                load_idx()
            else:
                @pl.when(g < ng - 1)
                def _():
                    load_idx()

            def wait_idx():
                pltpu.make_async_copy(dslice(i + 1), didx[b1],
                                      isem.at[b1]).wait()

            if k < 4:
                wait_idx()
            else:
                @pl.when(g < ng - 1)
                def _():
                    wait_idx()

            pltpu.async_copy(ones_v, cnt_sh.at[didx[b]], ssem.at[b],
                             add=True)
        return carry

    lax.fori_loop(0, ng, group, 0)
    for b in (2, 3, 4):
        pltpu.make_async_copy(ones_v, cnt_sh.at[didx[b]], ssem.at[b]).wait()
    plsc.subcore_barrier()
    _writeback(cnt_sh, fbuf, out_hbm, c, r0)


_SC_MESH = plsc.VectorSubcoreMesh(core_axis_name="c", subcore_axis_name="s",
                                  num_cores=NC, num_subcores=NS)

_agg_scratch = [
    pltpu.VMEM_SHARED((NPAD, D), jnp.float32),
    [pltpu.VMEM((K,), jnp.int32) for _ in range(RING)],
    [pltpu.VMEM((K,), jnp.int32) for _ in range(RING)],
    [pltpu.VMEM((K, D), jnp.float32) for _ in range(RING)],
    pltpu.VMEM((SB, D), jnp.float32),
    pltpu.SemaphoreType.DMA((RING,)),
    pltpu.SemaphoreType.DMA((RING,)),
    pltpu.SemaphoreType.DMA((RING,)),
]

_agg1 = pl.kernel(
    _agg1_body,
    out_type=jax.ShapeDtypeStruct((NC, NPAD, D), jnp.float32),
    mesh=_SC_MESH,
    scratch_types=_agg_scratch,
)

_agg2 = pl.kernel(
    _agg2_body,
    out_type=jax.ShapeDtypeStruct((NC, NPAD, D), jnp.float32),
    mesh=_SC_MESH,
    scratch_types=_agg_scratch,
)

_cnt = pl.kernel(
    _cnt_body,
    out_type=jax.ShapeDtypeStruct((NC, NPAD, 128), jnp.float32),
    mesh=_SC_MESH,
    scratch_types=[
        pltpu.VMEM_SHARED((NPAD, 128), jnp.float32),
        [pltpu.VMEM((K,), jnp.int32) for _ in range(RING)],
        pltpu.VMEM((K, 128), jnp.float32),
        pltpu.VMEM((SB, 128), jnp.float32),
        pltpu.SemaphoreType.DMA((RING,)),
        pltpu.SemaphoreType.DMA((RING,)),
    ],
)

R = 400


def _tc1_body(acc_ref, cnt_ref, x_ref, wl_ref, wr_ref, b_ref, out_ref):
    ssum = acc_ref[0] + acc_ref[1]
    cnt = cnt_ref[0, :, 0] + cnt_ref[1, :, 0]
    rc = 1.0 / jnp.maximum(cnt, 1.0)
    mean = ssum * rc[:, None]
    h = jnp.dot(mean, wl_ref[...], preferred_element_type=jnp.float32)
    h += jnp.dot(x_ref[...], wr_ref[...], preferred_element_type=jnp.float32)
    h += b_ref[...]
    h = jnp.maximum(h, 0.0)
    out_ref[0] = h[:, :D]
    out_ref[1] = h[:, D:]


_tc1 = pl.pallas_call(
    _tc1_body,
    grid=(N // R,),
    in_specs=[
        pl.BlockSpec((NC, R, D), lambda i: (0, i, 0)),
        pl.BlockSpec((NC, R, 128), lambda i: (0, i, 0)),
        pl.BlockSpec((R, D), lambda i: (i, 0)),
        pl.BlockSpec((D, H), lambda i: (0, 0)),
        pl.BlockSpec((D, H), lambda i: (0, 0)),
        pl.BlockSpec((1, H), lambda i: (0, 0)),
    ],
    out_specs=pl.BlockSpec((NC, R, D), lambda i: (0, i, 0)),
    out_shape=jax.ShapeDtypeStruct((NC, N, D), jnp.float32),
)


def _tc2_body(acc_ref, cnt_ref, h1_ref, batch_ref, wl_ref, wr_ref, b_ref,
              fcw_ref, fcb_ref, out_ref, pooled, cntg):
    i = pl.program_id(0)
    cnt = cnt_ref[0, :, 0] + cnt_ref[1, :, 0]
    rc = 1.0 / jnp.maximum(cnt, 1.0)
    wl = wl_ref[...]
    wr = wr_ref[...]
    h2 = jnp.dot(acc_ref[0] * rc[:, None], wl[:D],
                 preferred_element_type=jnp.float32)
    h2 += jnp.dot(acc_ref[1] * rc[:, None], wl[D:],
                  preferred_element_type=jnp.float32)
    h2 += jnp.dot(h1_ref[0], wr[:D], preferred_element_type=jnp.float32)
    h2 += jnp.dot(h1_ref[1], wr[D:], preferred_element_type=jnp.float32)
    h2 += b_ref[...]
    h2 = jnp.maximum(h2, 0.0)
    b = batch_ref[0, 0]
    onehot = (b[:, None] == lax.broadcasted_iota(jnp.int32, (R, G), 1)
              ).astype(jnp.float32)
    part = lax.dot_general(onehot, h2, (((0,), (0,)), ((), ())),
                           preferred_element_type=jnp.float32)
    cg_part = jnp.sum(onehot, axis=0)[None, :]

    @pl.when(i == 0)
    def _():
        pooled[...] = jnp.zeros_like(pooled)
        cntg[...] = jnp.zeros_like(cntg)

    pooled[...] += part
    cntg[...] += cg_part

    @pl.when(i == N // R - 1)
    def _():
        rcg = 1.0 / jnp.maximum(cntg[0, :], 1.0)
        pm = pooled[...] * rcg[:, None]
        out_ref[...] = (jnp.dot(pm, fcw_ref[...],
                                preferred_element_type=jnp.float32)
                        + fcb_ref[...])


_tc2 = pl.pallas_call(
    _tc2_body,
    grid=(N // R,),
    in_specs=[
        pl.BlockSpec((NC, R, D), lambda i: (0, i, 0)),
        pl.BlockSpec((NC, R, 128), lambda i: (0, i, 0)),
        pl.BlockSpec((NC, R, D), lambda i: (0, i, 0)),
        pl.BlockSpec((1, 1, R), lambda i: (i, 0, 0)),
        pl.BlockSpec((H, H), lambda i: (0, 0)),
        pl.BlockSpec((H, H), lambda i: (0, 0)),
        pl.BlockSpec((1, H), lambda i: (0, 0)),
        pl.BlockSpec((H, C), lambda i: (0, 0)),
        pl.BlockSpec((1, C), lambda i: (0, 0)),
    ],
    out_specs=pl.BlockSpec((G, C), lambda i: (0, 0)),
    out_shape=jax.ShapeDtypeStruct((G, C), jnp.float32),
    scratch_shapes=[
        pltpu.VMEM((G, H), jnp.float32),
        pltpu.VMEM((1, G), jnp.float32),
    ],
)


def kernel(x, edge_index, batch, Wl1, bl1, Wr1, Wl2, bl2, Wr2, fcW, fcb):
    src = edge_index[0]
    dst = edge_index[1]
    zf = jnp.zeros((SB, D), jnp.float32)
    ones = jnp.ones((K, 128), jnp.float32)
    acc1 = _agg1(x, src, dst, zf)
    cnth = _cnt(dst, ones, zf)
    h1 = _tc1(acc1, cnth, x, Wl1, Wr1, bl1.reshape(1, H))
    src2 = jnp.concatenate([src, src + N])
    acc2 = _agg2(h1.reshape(NC * N, D), src2, dst, zf)
    out = _tc2(acc2, cnth, h1, batch.reshape(N // R, 1, R),
               Wl2, Wr2, bl2.reshape(1, H), fcW, fcb.reshape(1, C))
    return out


# speedup vs baseline: 8.8360x; 1.1455x over previous
import jax
import jax.numpy as jnp
from jax import lax
from jax.experimental import pallas as pl
from jax.experimental.pallas import tpu as pltpu
from jax.experimental.pallas import tpu_sc as plsc

N = 10000
E = 320000
D = 128
H = 256
C = 10
G = 16

NC = 2
NS = 16
NW = NC * NS
K = 40
RING = 5
NPAD = 10240
RPT = NPAD // NS
SB = 32
EPT1 = E // NW
EPT2 = E // NS


def _zero_acc(zf_hbm, fbuf, acc_sh, r0):
    pltpu.sync_copy(zf_hbm, fbuf)
    for j in range(RPT // SB):
        pltpu.sync_copy(fbuf, acc_sh.at[pl.ds(r0 + j * SB, SB)])


def _writeback(acc_sh, fbuf, out_hbm, c, r0):
    for j in range(RPT // SB):
        pltpu.sync_copy(acc_sh.at[pl.ds(r0 + j * SB, SB)], fbuf)
        pltpu.sync_copy(fbuf, out_hbm.at[c, pl.ds(r0 + j * SB, SB)])


def _agg_pipeline(t_hbm, src_hbm, dst_hbm, acc_sh, sidx, didx, rows,
                  isem, gsem, ssem, base_s, base_d, ch):
    ng = ch // RING

    def sslice(i):
        return src_hbm.at[pl.ds(base_s + i * K, K)]

    def dslice(i):
        return dst_hbm.at[pl.ds(base_d + i * K, K)]

    for b in (0, 1, 2):
        pltpu.async_copy(sslice(b), sidx[b], isem.at[b])
        pltpu.async_copy(dslice(b), didx[b], isem.at[b])
    for b in (0, 1):
        pltpu.make_async_copy(sslice(b), sidx[b], isem.at[b]).wait()
        pltpu.make_async_copy(dslice(b), didx[b], isem.at[b]).wait()
        pltpu.async_copy(t_hbm.at[sidx[b]], rows[b], gsem.at[b])

    def group(g, carry):
        for k in range(RING):
            i = g * RING + k
            b = k
            b2 = (k + 2) % RING
            b3 = (k + 3) % RING

            def wait_s():
                pltpu.make_async_copy(rows[b3], acc_sh.at[didx[b3]],
                                      ssem.at[b3]).wait()

            if k >= 2:
                wait_s()
            else:
                @pl.when(g > 0)
                def _():
                    wait_s()

            def load_idx():
                pltpu.async_copy(sslice(i + 3), sidx[b3], isem.at[b3])
                pltpu.async_copy(dslice(i + 3), didx[b3], isem.at[b3])

            if k < 2:
                load_idx()
            else:
                @pl.when(g < ng - 1)
                def _():
                    load_idx()

            def start_gather():
                pltpu.make_async_copy(sslice(i + 2), sidx[b2],
                                      isem.at[b2]).wait()
                pltpu.make_async_copy(dslice(i + 2), didx[b2],
                                      isem.at[b2]).wait()
                pltpu.async_copy(t_hbm.at[sidx[b2]], rows[b2], gsem.at[b2])

            if k < 3:
                start_gather()
            else:
                @pl.when(g < ng - 1)
                def _():
                    start_gather()

            pltpu.make_async_copy(t_hbm.at[sidx[b]], rows[b],
                                  gsem.at[b]).wait()
            pltpu.async_copy(rows[b], acc_sh.at[didx[b]], ssem.at[b],
                             add=True)
        return carry

    lax.fori_loop(0, ng, group, 0)
    for b in (3, 4):
        pltpu.make_async_copy(rows[b], acc_sh.at[didx[b]], ssem.at[b]).wait()


def _agg1_body(x_hbm, src_hbm, dst_hbm, zf_hbm, out_hbm,
               acc_sh, sidx, didx, rows, fbuf, isem, gsem, ssem):
    c = lax.axis_index("c")
    s = lax.axis_index("s")
    r0 = s * RPT
    _zero_acc(zf_hbm, fbuf, acc_sh, r0)
    plsc.subcore_barrier()
    base = (c * NS + s) * EPT1
    _agg_pipeline(x_hbm, src_hbm, dst_hbm, acc_sh, sidx, didx, rows,
                  isem, gsem, ssem, base, base, EPT1 // K)
    plsc.subcore_barrier()
    _writeback(acc_sh, fbuf, out_hbm, c, r0)


def _agg2_body(t_hbm, src2_hbm, dst_hbm, zf_hbm, out_hbm,
               acc_sh, sidx, didx, rows, fbuf, isem, gsem, ssem):
    c = lax.axis_index("c")
    s = lax.axis_index("s")
    r0 = s * RPT
    _zero_acc(zf_hbm, fbuf, acc_sh, r0)
    plsc.subcore_barrier()
    _agg_pipeline(t_hbm, src2_hbm, dst_hbm, acc_sh, sidx, didx, rows,
                  isem, gsem, ssem, c * E + s * EPT2, s * EPT2, EPT2 // K)
    plsc.subcore_barrier()
    _writeback(acc_sh, fbuf, out_hbm, c, r0)


def _cnt_body(dst_hbm, ones_hbm, zf_hbm, out_hbm,
              cnt_sh, didx, ones_v, fbuf, isem, ssem):
    c = lax.axis_index("c")
    s = lax.axis_index("s")
    r0 = s * RPT
    _zero_acc(zf_hbm, fbuf, cnt_sh, r0)
    pltpu.sync_copy(ones_hbm, ones_v)
    plsc.subcore_barrier()
    base = (c * NS + s) * EPT1
    ch = EPT1 // K
    ng = ch // RING

    def dslice(i):
        return dst_hbm.at[pl.ds(base + i * K, K)]

    for b in (0, 1, 2):
        pltpu.async_copy(dslice(b), didx[b], isem.at[b])

    def group(g, carry):
        for k in range(RING):
            i = g * RING + k
            b = k
            b3 = (k + 3) % RING

            def wait_s():
                pltpu.make_async_copy(ones_v, cnt_sh.at[didx[b3]],
                                      ssem.at[b3]).wait()

            if k >= 2:
                wait_s()
            else:
                @pl.when(g > 0)
                def _():
                    wait_s()

            def load_idx():
                pltpu.async_copy(dslice(i + 3), didx[b3], isem.at[b3])

            if k < 2:
                load_idx()
            else:
                @pl.when(g < ng - 1)
                def _():
                    load_idx()

            pltpu.make_async_copy(dslice(i), didx[b], isem.at[b]).wait()
            pltpu.async_copy(ones_v, cnt_sh.at[didx[b]], ssem.at[b],
                             add=True)
        return carry

    lax.fori_loop(0, ng, group, 0)
    for b in (3, 4):
        pltpu.make_async_copy(ones_v, cnt_sh.at[didx[b]], ssem.at[b]).wait()
    plsc.subcore_barrier()
    _writeback(cnt_sh, fbuf, out_hbm, c, r0)


_SC_MESH = plsc.VectorSubcoreMesh(core_axis_name="c", subcore_axis_name="s",
                                  num_cores=NC, num_subcores=NS)

_agg_scratch = [
    pltpu.VMEM_SHARED((NPAD, D), jnp.float32),
    [pltpu.VMEM((K,), jnp.int32) for _ in range(RING)],
    [pltpu.VMEM((K,), jnp.int32) for _ in range(RING)],
    [pltpu.VMEM((K, D), jnp.float32) for _ in range(RING)],
    pltpu.VMEM((SB, D), jnp.float32),
    pltpu.SemaphoreType.DMA((RING,)),
    pltpu.SemaphoreType.DMA((RING,)),
    pltpu.SemaphoreType.DMA((RING,)),
]

_agg1 = pl.kernel(
    _agg1_body,
    out_type=jax.ShapeDtypeStruct((NC, NPAD, D), jnp.float32),
    mesh=_SC_MESH,
    scratch_types=_agg_scratch,
)

_agg2 = pl.kernel(
    _agg2_body,
    out_type=jax.ShapeDtypeStruct((NC, NPAD, D), jnp.float32),
    mesh=_SC_MESH,
    scratch_types=_agg_scratch,
)

_cnt = pl.kernel(
    _cnt_body,
    out_type=jax.ShapeDtypeStruct((NC, NPAD, 128), jnp.float32),
    mesh=_SC_MESH,
    scratch_types=[
        pltpu.VMEM_SHARED((NPAD, 128), jnp.float32),
        [pltpu.VMEM((K,), jnp.int32) for _ in range(RING)],
        pltpu.VMEM((K, 128), jnp.float32),
        pltpu.VMEM((SB, 128), jnp.float32),
        pltpu.SemaphoreType.DMA((RING,)),
        pltpu.SemaphoreType.DMA((RING,)),
    ],
)

R = 400


def _tc1_body(acc_ref, cnt_ref, x_ref, wl_ref, wr_ref, b_ref, out_ref):
    ssum = acc_ref[0] + acc_ref[1]
    cnt = cnt_ref[0, :, 0] + cnt_ref[1, :, 0]
    rc = 1.0 / jnp.maximum(cnt, 1.0)
    mean = ssum * rc[:, None]
    h = jnp.dot(mean, wl_ref[...], preferred_element_type=jnp.float32)
    h += jnp.dot(x_ref[...], wr_ref[...], preferred_element_type=jnp.float32)
    h += b_ref[...]
    h = jnp.maximum(h, 0.0)
    out_ref[0] = h[:, :D]
    out_ref[1] = h[:, D:]


_tc1 = pl.pallas_call(
    _tc1_body,
    grid=(N // R,),
    in_specs=[
        pl.BlockSpec((NC, R, D), lambda i: (0, i, 0)),
        pl.BlockSpec((NC, R, 128), lambda i: (0, i, 0)),
        pl.BlockSpec((R, D), lambda i: (i, 0)),
        pl.BlockSpec((D, H), lambda i: (0, 0)),
        pl.BlockSpec((D, H), lambda i: (0, 0)),
        pl.BlockSpec((1, H), lambda i: (0, 0)),
    ],
    out_specs=pl.BlockSpec((NC, R, D), lambda i: (0, i, 0)),
    out_shape=jax.ShapeDtypeStruct((NC, N, D), jnp.float32),
)


def _tc2_body(acc_ref, cnt_ref, h1_ref, batch_ref, wl_ref, wr_ref, b_ref,
              fcw_ref, fcb_ref, out_ref, pooled, cntg):
    i = pl.program_id(0)
    cnt = cnt_ref[0, :, 0] + cnt_ref[1, :, 0]
    rc = 1.0 / jnp.maximum(cnt, 1.0)
    wl = wl_ref[...]
    wr = wr_ref[...]
    h2 = jnp.dot(acc_ref[0] * rc[:, None], wl[:D],
                 preferred_element_type=jnp.float32)
    h2 += jnp.dot(acc_ref[1] * rc[:, None], wl[D:],
                  preferred_element_type=jnp.float32)
    h2 += jnp.dot(h1_ref[0], wr[:D], preferred_element_type=jnp.float32)
    h2 += jnp.dot(h1_ref[1], wr[D:], preferred_element_type=jnp.float32)
    h2 += b_ref[...]
    h2 = jnp.maximum(h2, 0.0)
    b = batch_ref[0, 0]
    onehot = (b[:, None] == lax.broadcasted_iota(jnp.int32, (R, G), 1)
              ).astype(jnp.float32)
    part = lax.dot_general(onehot, h2, (((0,), (0,)), ((), ())),
                           preferred_element_type=jnp.float32)
    cg_part = jnp.sum(onehot, axis=0)[None, :]

    @pl.when(i == 0)
    def _():
        pooled[...] = jnp.zeros_like(pooled)
        cntg[...] = jnp.zeros_like(cntg)

    pooled[...] += part
    cntg[...] += cg_part

    @pl.when(i == N // R - 1)
    def _():
        rcg = 1.0 / jnp.maximum(cntg[0, :], 1.0)
        pm = pooled[...] * rcg[:, None]
        out_ref[...] = (jnp.dot(pm, fcw_ref[...],
                                preferred_element_type=jnp.float32)
                        + fcb_ref[...])


_tc2 = pl.pallas_call(
    _tc2_body,
    grid=(N // R,),
    in_specs=[
        pl.BlockSpec((NC, R, D), lambda i: (0, i, 0)),
        pl.BlockSpec((NC, R, 128), lambda i: (0, i, 0)),
        pl.BlockSpec((NC, R, D), lambda i: (0, i, 0)),
        pl.BlockSpec((1, 1, R), lambda i: (i, 0, 0)),
        pl.BlockSpec((H, H), lambda i: (0, 0)),
        pl.BlockSpec((H, H), lambda i: (0, 0)),
        pl.BlockSpec((1, H), lambda i: (0, 0)),
        pl.BlockSpec((H, C), lambda i: (0, 0)),
        pl.BlockSpec((1, C), lambda i: (0, 0)),
    ],
    out_specs=pl.BlockSpec((G, C), lambda i: (0, 0)),
    out_shape=jax.ShapeDtypeStruct((G, C), jnp.float32),
    scratch_shapes=[
        pltpu.VMEM((G, H), jnp.float32),
        pltpu.VMEM((1, G), jnp.float32),
    ],
)


def kernel(x, edge_index, batch, Wl1, bl1, Wr1, Wl2, bl2, Wr2, fcW, fcb):
    src = edge_index[0]
    dst = edge_index[1]
    zf = jnp.zeros((SB, D), jnp.float32)
    ones = jnp.ones((K, 128), jnp.float32)
    acc1 = _agg1(x, src, dst, zf)
    cnth = _cnt(dst, ones, zf)
    h1 = _tc1(acc1, cnth, x, Wl1, Wr1, bl1.reshape(1, H))
    src2 = jnp.concatenate([src, src + N])
    acc2 = _agg2(h1.reshape(NC * N, D), src2, dst, zf)
    out = _tc2(acc2, cnth, h1, batch.reshape(N // R, 1, R),
               Wl2, Wr2, bl2.reshape(1, H), fcW, fcb.reshape(1, C))
    return out


# speedup vs baseline: 8.9330x; 1.0110x over previous
import jax
import jax.numpy as jnp
from jax import lax
from jax.experimental import pallas as pl
from jax.experimental.pallas import tpu as pltpu
from jax.experimental.pallas import tpu_sc as plsc

N = 10000
E = 320000
D = 128
H = 256
C = 10
G = 16

NC = 2
NS = 16
NW = NC * NS
K = 40
RING = 5
NPAD = 10240
RPT = NPAD // NS
SB = 32
EPT1 = E // NW
EPT2 = E // NS


def _zero_acc(zf_hbm, fbuf, acc_sh, r0):
    pltpu.sync_copy(zf_hbm, fbuf)
    for j in range(RPT // SB):
        pltpu.sync_copy(fbuf, acc_sh.at[pl.ds(r0 + j * SB, SB)])


def _writeback(acc_sh, fbuf, out_hbm, c, r0):
    for j in range(RPT // SB):
        pltpu.sync_copy(acc_sh.at[pl.ds(r0 + j * SB, SB)], fbuf)
        pltpu.sync_copy(fbuf, out_hbm.at[c, pl.ds(r0 + j * SB, SB)])


def _agg_pipeline(t_hbm, src_hbm, dst_hbm, acc_sh, sidx, didx, rows,
                  isem, gsem, ssem, base_s, base_d, ch):
    ng = ch // RING

    def sslice(i):
        return src_hbm.at[pl.ds(base_s + i * K, K)]

    def dslice(i):
        return dst_hbm.at[pl.ds(base_d + i * K, K)]

    for b in (0, 1, 2):
        pltpu.async_copy(sslice(b), sidx[b], isem.at[b])
        pltpu.async_copy(dslice(b), didx[b], isem.at[b])
    for b in (0, 1):
        pltpu.make_async_copy(sslice(b), sidx[b], isem.at[b]).wait()
        pltpu.make_async_copy(dslice(b), didx[b], isem.at[b]).wait()
        pltpu.async_copy(t_hbm.at[sidx[b]], rows[b], gsem.at[b])

    def group(g, carry):
        for k in range(RING):
            i = g * RING + k
            b = k
            b2 = (k + 2) % RING
            b3 = (k + 3) % RING

            def wait_s():
                pltpu.make_async_copy(rows[b3], acc_sh.at[didx[b3]],
                                      ssem.at[b3]).wait()

            if k >= 2:
                wait_s()
            else:
                @pl.when(g > 0)
                def _():
                    wait_s()

            def load_idx():
                pltpu.async_copy(sslice(i + 3), sidx[b3], isem.at[b3])
                pltpu.async_copy(dslice(i + 3), didx[b3], isem.at[b3])

            if k < 2:
                load_idx()
            else:
                @pl.when(g < ng - 1)
                def _():
                    load_idx()

            def start_gather():
                pltpu.make_async_copy(sslice(i + 2), sidx[b2],
                                      isem.at[b2]).wait()
                pltpu.make_async_copy(dslice(i + 2), didx[b2],
                                      isem.at[b2]).wait()
                pltpu.async_copy(t_hbm.at[sidx[b2]], rows[b2], gsem.at[b2])

            if k < 3:
                start_gather()
            else:
                @pl.when(g < ng - 1)
                def _():
                    start_gather()

            pltpu.make_async_copy(t_hbm.at[sidx[b]], rows[b],
                                  gsem.at[b]).wait()
            pltpu.async_copy(rows[b], acc_sh.at[didx[b]], ssem.at[b],
                             add=True)
        return carry

    lax.fori_loop(0, ng, group, 0)
    for b in (3, 4):
        pltpu.make_async_copy(rows[b], acc_sh.at[didx[b]], ssem.at[b]).wait()


def _cnt_pipeline(dst_hbm, cnt_sh, didx, ones_v, isem, ssem, base, ch):
    ng = ch // RING

    def dslice(i):
        return dst_hbm.at[pl.ds(base + i * K, K)]

    for b in (0, 1, 2):
        pltpu.async_copy(dslice(b), didx[b], isem.at[b])

    def group(g, carry):
        for k in range(RING):
            i = g * RING + k
            b = k
            b3 = (k + 3) % RING

            def wait_s():
                pltpu.make_async_copy(ones_v, cnt_sh.at[didx[b3]],
                                      ssem.at[b3]).wait()

            if k >= 2:
                wait_s()
            else:
                @pl.when(g > 0)
                def _():
                    wait_s()

            def load_idx():
                pltpu.async_copy(dslice(i + 3), didx[b3], isem.at[b3])

            if k < 2:
                load_idx()
            else:
                @pl.when(g < ng - 1)
                def _():
                    load_idx()

            pltpu.make_async_copy(dslice(i), didx[b], isem.at[b]).wait()
            pltpu.async_copy(ones_v, cnt_sh.at[didx[b]], ssem.at[b],
                             add=True)
        return carry

    lax.fori_loop(0, ng, group, 0)
    for b in (3, 4):
        pltpu.make_async_copy(ones_v, cnt_sh.at[didx[b]], ssem.at[b]).wait()


def _agg1_body(x_hbm, src_hbm, dst_hbm, zf_hbm, ones_hbm, out_hbm, cnt_hbm,
               acc_sh, sidx, didx, rows, ones_v, fbuf, isem, gsem, ssem):
    c = lax.axis_index("c")
    s = lax.axis_index("s")
    r0 = s * RPT
    _zero_acc(zf_hbm, fbuf, acc_sh, r0)
    pltpu.sync_copy(ones_hbm, ones_v)
    plsc.subcore_barrier()
    base = (c * NS + s) * EPT1
    _agg_pipeline(x_hbm, src_hbm, dst_hbm, acc_sh, sidx, didx, rows,
                  isem, gsem, ssem, base, base, EPT1 // K)
    plsc.subcore_barrier()
    _writeback(acc_sh, fbuf, out_hbm, c, r0)
    _zero_acc(zf_hbm, fbuf, acc_sh, r0)
    plsc.subcore_barrier()
    _cnt_pipeline(dst_hbm, acc_sh, didx, ones_v, isem, ssem, base, EPT1 // K)
    plsc.subcore_barrier()
    _writeback(acc_sh, fbuf, cnt_hbm, c, r0)


def _agg2_body(t_hbm, src2_hbm, dst_hbm, zf_hbm, out_hbm,
               acc_sh, sidx, didx, rows, fbuf, isem, gsem, ssem):
    c = lax.axis_index("c")
    s = lax.axis_index("s")
    r0 = s * RPT
    _zero_acc(zf_hbm, fbuf, acc_sh, r0)
    plsc.subcore_barrier()
    _agg_pipeline(t_hbm, src2_hbm, dst_hbm, acc_sh, sidx, didx, rows,
                  isem, gsem, ssem, c * E + s * EPT2, s * EPT2, EPT2 // K)
    plsc.subcore_barrier()
    _writeback(acc_sh, fbuf, out_hbm, c, r0)


def _cnt_body(dst_hbm, ones_hbm, zf_hbm, out_hbm,
              cnt_sh, didx, ones_v, fbuf, isem, ssem):
    c = lax.axis_index("c")
    s = lax.axis_index("s")
    r0 = s * RPT
    _zero_acc(zf_hbm, fbuf, cnt_sh, r0)
    pltpu.sync_copy(ones_hbm, ones_v)
    plsc.subcore_barrier()
    base = (c * NS + s) * EPT1
    ch = EPT1 // K
    ng = ch // RING

    def dslice(i):
        return dst_hbm.at[pl.ds(base + i * K, K)]

    for b in (0, 1, 2):
        pltpu.async_copy(dslice(b), didx[b], isem.at[b])

    def group(g, carry):
        for k in range(RING):
            i = g * RING + k
            b = k
            b3 = (k + 3) % RING

            def wait_s():
                pltpu.make_async_copy(ones_v, cnt_sh.at[didx[b3]],
                                      ssem.at[b3]).wait()

            if k >= 2:
                wait_s()
            else:
                @pl.when(g > 0)
                def _():
                    wait_s()

            def load_idx():
                pltpu.async_copy(dslice(i + 3), didx[b3], isem.at[b3])

            if k < 2:
                load_idx()
            else:
                @pl.when(g < ng - 1)
                def _():
                    load_idx()

            pltpu.make_async_copy(dslice(i), didx[b], isem.at[b]).wait()
            pltpu.async_copy(ones_v, cnt_sh.at[didx[b]], ssem.at[b],
                             add=True)
        return carry

    lax.fori_loop(0, ng, group, 0)
    for b in (3, 4):
        pltpu.make_async_copy(ones_v, cnt_sh.at[didx[b]], ssem.at[b]).wait()
    plsc.subcore_barrier()
    _writeback(cnt_sh, fbuf, out_hbm, c, r0)


_SC_MESH = plsc.VectorSubcoreMesh(core_axis_name="c", subcore_axis_name="s",
                                  num_cores=NC, num_subcores=NS)

_agg_scratch = [
    pltpu.VMEM_SHARED((NPAD, D), jnp.float32),
    [pltpu.VMEM((K,), jnp.int32) for _ in range(RING)],
    [pltpu.VMEM((K,), jnp.int32) for _ in range(RING)],
    [pltpu.VMEM((K, D), jnp.float32) for _ in range(RING)],
    pltpu.VMEM((SB, D), jnp.float32),
    pltpu.SemaphoreType.DMA((RING,)),
    pltpu.SemaphoreType.DMA((RING,)),
    pltpu.SemaphoreType.DMA((RING,)),
]

_agg1 = pl.kernel(
    _agg1_body,
    out_type=(jax.ShapeDtypeStruct((NC, NPAD, D), jnp.float32),
              jax.ShapeDtypeStruct((NC, NPAD, 128), jnp.float32)),
    mesh=_SC_MESH,
    scratch_types=[
        pltpu.VMEM_SHARED((NPAD, D), jnp.float32),
        [pltpu.VMEM((K,), jnp.int32) for _ in range(RING)],
        [pltpu.VMEM((K,), jnp.int32) for _ in range(RING)],
        [pltpu.VMEM((K, D), jnp.float32) for _ in range(RING)],
        pltpu.VMEM((K, 128), jnp.float32),
        pltpu.VMEM((SB, D), jnp.float32),
        pltpu.SemaphoreType.DMA((RING,)),
        pltpu.SemaphoreType.DMA((RING,)),
        pltpu.SemaphoreType.DMA((RING,)),
    ],
)

_agg2 = pl.kernel(
    _agg2_body,
    out_type=jax.ShapeDtypeStruct((NC, NPAD, D), jnp.float32),
    mesh=_SC_MESH,
    scratch_types=_agg_scratch,
)

R = 400


def _tc1_body(acc_ref, cnt_ref, x_ref, wl_ref, wr_ref, b_ref, out_ref):
    ssum = acc_ref[0] + acc_ref[1]
    cnt = cnt_ref[0, :, 0] + cnt_ref[1, :, 0]
    rc = 1.0 / jnp.maximum(cnt, 1.0)
    mean = ssum * rc[:, None]
    h = jnp.dot(mean, wl_ref[...], preferred_element_type=jnp.float32)
    h += jnp.dot(x_ref[...], wr_ref[...], preferred_element_type=jnp.float32)
    h += b_ref[...]
    h = jnp.maximum(h, 0.0)
    out_ref[0] = h[:, :D]
    out_ref[1] = h[:, D:]


_tc1 = pl.pallas_call(
    _tc1_body,
    grid=(N // R,),
    in_specs=[
        pl.BlockSpec((NC, R, D), lambda i: (0, i, 0)),
        pl.BlockSpec((NC, R, 128), lambda i: (0, i, 0)),
        pl.BlockSpec((R, D), lambda i: (i, 0)),
        pl.BlockSpec((D, H), lambda i: (0, 0)),
        pl.BlockSpec((D, H), lambda i: (0, 0)),
        pl.BlockSpec((1, H), lambda i: (0, 0)),
    ],
    out_specs=pl.BlockSpec((NC, R, D), lambda i: (0, i, 0)),
    out_shape=jax.ShapeDtypeStruct((NC, N, D), jnp.float32),
)


def _tc2_body(acc_ref, cnt_ref, h1_ref, batch_ref, wl_ref, wr_ref, b_ref,
              fcw_ref, fcb_ref, out_ref, pooled, cntg):
    i = pl.program_id(0)
    cnt = cnt_ref[0, :, 0] + cnt_ref[1, :, 0]
    rc = 1.0 / jnp.maximum(cnt, 1.0)
    wl = wl_ref[...]
    wr = wr_ref[...]
    h2 = jnp.dot(acc_ref[0] * rc[:, None], wl[:D],
                 preferred_element_type=jnp.float32)
    h2 += jnp.dot(acc_ref[1] * rc[:, None], wl[D:],
                  preferred_element_type=jnp.float32)
    h2 += jnp.dot(h1_ref[0], wr[:D], preferred_element_type=jnp.float32)
    h2 += jnp.dot(h1_ref[1], wr[D:], preferred_element_type=jnp.float32)
    h2 += b_ref[...]
    h2 = jnp.maximum(h2, 0.0)
    b = batch_ref[0, 0]
    onehot = (b[:, None] == lax.broadcasted_iota(jnp.int32, (R, G), 1)
              ).astype(jnp.float32)
    part = lax.dot_general(onehot, h2, (((0,), (0,)), ((), ())),
                           preferred_element_type=jnp.float32)
    cg_part = jnp.sum(onehot, axis=0)[None, :]

    @pl.when(i == 0)
    def _():
        pooled[...] = jnp.zeros_like(pooled)
        cntg[...] = jnp.zeros_like(cntg)

    pooled[...] += part
    cntg[...] += cg_part

    @pl.when(i == N // R - 1)
    def _():
        rcg = 1.0 / jnp.maximum(cntg[0, :], 1.0)
        pm = pooled[...] * rcg[:, None]
        out_ref[...] = (jnp.dot(pm, fcw_ref[...],
                                preferred_element_type=jnp.float32)
                        + fcb_ref[...])


_tc2 = pl.pallas_call(
    _tc2_body,
    grid=(N // R,),
    in_specs=[
        pl.BlockSpec((NC, R, D), lambda i: (0, i, 0)),
        pl.BlockSpec((NC, R, 128), lambda i: (0, i, 0)),
        pl.BlockSpec((NC, R, D), lambda i: (0, i, 0)),
        pl.BlockSpec((1, 1, R), lambda i: (i, 0, 0)),
        pl.BlockSpec((H, H), lambda i: (0, 0)),
        pl.BlockSpec((H, H), lambda i: (0, 0)),
        pl.BlockSpec((1, H), lambda i: (0, 0)),
        pl.BlockSpec((H, C), lambda i: (0, 0)),
        pl.BlockSpec((1, C), lambda i: (0, 0)),
    ],
    out_specs=pl.BlockSpec((G, C), lambda i: (0, 0)),
    out_shape=jax.ShapeDtypeStruct((G, C), jnp.float32),
    scratch_shapes=[
        pltpu.VMEM((G, H), jnp.float32),
        pltpu.VMEM((1, G), jnp.float32),
    ],
)


def kernel(x, edge_index, batch, Wl1, bl1, Wr1, Wl2, bl2, Wr2, fcW, fcb):
    src = edge_index[0]
    dst = edge_index[1]
    zf = jnp.zeros((SB, D), jnp.float32)
    ones = jnp.ones((K, 128), jnp.float32)
    acc1, cnth = _agg1(x, src, dst, zf, ones)
    h1 = _tc1(acc1, cnth, x, Wl1, Wr1, bl1.reshape(1, H))
    src2 = jnp.concatenate([src, src + N])
    acc2 = _agg2(h1.reshape(NC * N, D), src2, dst, zf)
    out = _tc2(acc2, cnth, h1, batch.reshape(N // R, 1, R),
               Wl2, Wr2, bl2.reshape(1, H), fcW, fcb.reshape(1, C))
    return out
